# baseline (speedup 1.0000x reference)
import functools

import jax
import jax.numpy as jnp
from jax.experimental import pallas as pl
from jax.experimental.pallas import tpu as pltpu

_K = 50
_DELTA = 1e-3
_W = 2048
_BQ = 64
_PAD = 3.0e38


def _ksq_kernel(kT_ref, out_ref, *, C, W):
    j = pl.program_id(0)
    kT = kT_ref[...]
    ksq = jnp.sum(kT * kT, axis=0, keepdims=True)
    col = jax.lax.broadcasted_iota(jnp.int32, ksq.shape, 1) + j * W
    out_ref[...] = jnp.where(col < C, ksq, jnp.float32(_PAD))


def _dnd_kernel(q_ref, kT_ref, ksq_ref, vals_ref, out_ref, d_scr,
                *, C_pad, NT):
    q = q_ref[...]
    qsq = jnp.sum(q * q, axis=1, keepdims=True)

    def mm(tt, carry):
        kT = kT_ref[:, pl.ds(tt * _W, _W)]
        prod = jnp.dot(q, kT, preferred_element_type=jnp.float32)
        d = (qsq - 2.0 * prod) + ksq_ref[:, pl.ds(tt * _W, _W)]
        d_scr[:, pl.ds(tt * _W, _W)] = jnp.maximum(d, jnp.float32(0.0))
        return carry

    jax.lax.fori_loop(0, NT, mm, 0)

    if True:
        Bq = q.shape[0]

        def count_le(tf):
            def tb(tt, acc):
                tile = d_scr[:, pl.ds(tt * _W, _W)]
                return acc + jnp.sum((tile <= tf).astype(jnp.int32), axis=1,
                                     keepdims=True)
            return jax.lax.fori_loop(0, NT, tb,
                                     jnp.zeros((Bq, 1), jnp.int32))

        def rowbounds():
            def tb(tt, lh):
                lo, hi = lh
                tile = d_scr[:, pl.ds(tt * _W, _W)]
                return (jnp.minimum(lo, jnp.min(tile, axis=1, keepdims=True)),
                        jnp.minimum(hi, jnp.max(tile, axis=1, keepdims=True)))
            init = (jnp.full((Bq, 1), jnp.float32(_PAD)),
                    jnp.full((Bq, 1), jnp.float32(_PAD)))
            return jax.lax.fori_loop(0, NT, tb, init)

        lo0f, hi0f = rowbounds()
        lo0 = jax.lax.bitcast_convert_type(lo0f, jnp.int32)
        hi0 = jax.lax.bitcast_convert_type(hi0f, jnp.int32)

        def bs_cond(lh):
            lo, hi = lh
            return jnp.any(lo < hi)

        def bs_body(lh):
            lo, hi = lh
            mid = (lo >> 1) + (hi >> 1) + (lo & hi & 1)
            midf = jax.lax.bitcast_convert_type(mid, jnp.float32)
            pred = count_le(midf) >= _K
            return (jnp.where(pred, lo, mid + 1), jnp.where(pred, mid, hi))

        _, Ti = jax.lax.while_loop(bs_cond, bs_body, (lo0, hi0))
        T = jax.lax.bitcast_convert_type(Ti, jnp.float32)
        Tlt = jax.lax.bitcast_convert_type(Ti - 1, jnp.float32)

        cnt_le = count_le(T)
        cnt_lt = count_le(Tlt)
        m = _K - cnt_lt

        def count_tie_le(jv):
            def tb(tt, acc):
                tile = d_scr[:, pl.ds(tt * _W, _W)]
                cid = jax.lax.broadcasted_iota(jnp.int32, tile.shape, 1) \
                    + tt * _W
                msk = (tile == T) & (cid <= jv)
                return acc + jnp.sum(msk.astype(jnp.int32), axis=1,
                                     keepdims=True)
            return jax.lax.fori_loop(0, NT, tb,
                                     jnp.zeros((Bq, 1), jnp.int32))

        J_all = jnp.full((Bq, 1), jnp.int32(C_pad))

        def idx_search(_):
            def b2(_, lh):
                lo, hi = lh
                mid = (lo + hi) >> 1
                p = count_tie_le(mid) >= m
                return (jnp.where(p, lo, mid + 1), jnp.where(p, mid, hi))
            nbits = max(1, (C_pad).bit_length())
            _, hi2 = jax.lax.fori_loop(
                0, nbits, b2, (jnp.zeros((Bq, 1), jnp.int32), J_all))
            return hi2

        J = jax.lax.cond(jnp.any(cnt_le > _K), idx_search,
                         lambda _: J_all, None)

        def agg(tt, acc):
            sw, swv = acc
            tile = d_scr[:, pl.ds(tt * _W, _W)]
            cid = jax.lax.broadcasted_iota(jnp.int32, tile.shape, 1) + tt * _W
            sel = (tile < T) | ((tile == T) & (cid <= J))
            w = 1.0 / (tile + jnp.float32(_DELTA))
            w = jnp.where(sel, w, 0.0)
            v = vals_ref[:, pl.ds(tt * _W, _W)]
            return (sw + jnp.sum(w, axis=1, keepdims=True),
                    swv + jnp.sum(w * v, axis=1, keepdims=True))

        z = jnp.zeros((Bq, 1), jnp.float32)
        sw, swv = jax.lax.fori_loop(0, NT, agg, (z, z))
        out_ref[...] = (swv / sw).reshape(1, 1, Bq)


def kernel(keys, dnd_keys, dnd_values):
    B, D = keys.shape
    C, _ = dnd_keys.shape
    NT = (C + _W - 1) // _W
    C_pad = NT * _W
    NB = (B + _BQ - 1) // _BQ
    B_pad = NB * _BQ

    kT = jnp.zeros((D, C_pad), jnp.float32).at[:, :C].set(dnd_keys.T)
    vals = jnp.zeros((1, C_pad), jnp.float32).at[0, :C].set(dnd_values)
    q = keys
    if B_pad != B:
        q = jnp.zeros((B_pad, D), jnp.float32).at[:B].set(keys)

    ksq = pl.pallas_call(
        functools.partial(_ksq_kernel, C=C, W=_W),
        grid=(NT,),
        in_specs=[pl.BlockSpec((D, _W), lambda j: (0, j))],
        out_specs=pl.BlockSpec((1, _W), lambda j: (0, j)),
        out_shape=jax.ShapeDtypeStruct((1, C_pad), jnp.float32),
    )(kT)

    out = pl.pallas_call(
        functools.partial(_dnd_kernel, C_pad=C_pad, NT=NT),
        grid=(NB,),
        in_specs=[
            pl.BlockSpec((_BQ, D), lambda i: (i, 0)),
            pl.BlockSpec((D, C_pad), lambda i: (0, 0)),
            pl.BlockSpec((1, C_pad), lambda i: (0, 0)),
            pl.BlockSpec((1, C_pad), lambda i: (0, 0)),
        ],
        out_specs=pl.BlockSpec((1, 1, _BQ), lambda i: (i, 0, 0)),
        out_shape=jax.ShapeDtypeStruct((NB, 1, _BQ), jnp.float32),
        scratch_shapes=[pltpu.VMEM((_BQ, C_pad), jnp.float32)],
        compiler_params=pltpu.CompilerParams(
            dimension_semantics=("parallel",),
            vmem_limit_bytes=63 * 1024 * 1024),
    )(q, kT, ksq, vals)
    return out.reshape(-1)[:B]


# speedup vs baseline: 1.3110x; 1.3110x over previous
import functools

import jax
import jax.numpy as jnp
from jax.experimental import pallas as pl
from jax.experimental.pallas import tpu as pltpu

_K = 50
_DELTA = 1e-3
_W = 2048
_BQ = 128
_PAD = 3.0e38


def _ksq_kernel(kT_ref, out_ref, *, C, W):
    j = pl.program_id(0)
    kT = kT_ref[...]
    ksq = jnp.sum(kT * kT, axis=0, keepdims=True)
    col = jax.lax.broadcasted_iota(jnp.int32, ksq.shape, 1) + j * W
    out_ref[...] = jnp.where(col < C, ksq, jnp.float32(_PAD))


def _dnd_kernel(q_ref, kT_ref, ksq_ref, vals_ref, out_ref, d_scr, qsq_scr,
                *, C_pad, NT):
    j = pl.program_id(1)
    q = q_ref[...]

    @pl.when(j == 0)
    def _prep():
        qsq_scr[...] = jnp.sum(q * q, axis=1, keepdims=True)

    kT = kT_ref[...]
    prod = jnp.dot(q, kT, preferred_element_type=jnp.float32)
    d = (qsq_scr[...] - 2.0 * prod) + ksq_ref[:, pl.ds(j * _W, _W)]
    d_scr[:, pl.ds(j * _W, _W)] = jnp.maximum(d, jnp.float32(0.0))

    @pl.when(j == NT - 1)
    def _finalize():
        Bq = q.shape[0]

        def count_le(tf):
            def tb(tt, acc):
                tile = d_scr[:, pl.ds(tt * _W, _W)]
                return acc + jnp.sum((tile <= tf).astype(jnp.int32), axis=1,
                                     keepdims=True)
            return jax.lax.fori_loop(0, NT, tb,
                                     jnp.zeros((Bq, 1), jnp.int32))

        def rowbounds():
            def tb(tt, lh):
                lo, hi = lh
                tile = d_scr[:, pl.ds(tt * _W, _W)]
                return (jnp.minimum(lo, jnp.min(tile, axis=1, keepdims=True)),
                        jnp.minimum(hi, jnp.max(tile, axis=1, keepdims=True)))
            init = (jnp.full((Bq, 1), jnp.float32(_PAD)),
                    jnp.full((Bq, 1), jnp.float32(_PAD)))
            return jax.lax.fori_loop(0, NT, tb, init)

        lo0f, hi0f = rowbounds()
        lo0 = jax.lax.bitcast_convert_type(lo0f, jnp.int32)
        hi0 = jax.lax.bitcast_convert_type(hi0f, jnp.int32)

        def bs_cond(lh):
            lo, hi = lh
            return jnp.any(lo < hi)

        def bs_body(lh):
            lo, hi = lh
            mid = (lo >> 1) + (hi >> 1) + (lo & hi & 1)
            midf = jax.lax.bitcast_convert_type(mid, jnp.float32)
            pred = count_le(midf) >= _K
            return (jnp.where(pred, lo, mid + 1), jnp.where(pred, mid, hi))

        _, Ti = jax.lax.while_loop(bs_cond, bs_body, (lo0, hi0))
        T = jax.lax.bitcast_convert_type(Ti, jnp.float32)
        Tlt = jax.lax.bitcast_convert_type(Ti - 1, jnp.float32)

        cnt_le = count_le(T)
        cnt_lt = count_le(Tlt)
        m = _K - cnt_lt

        def count_tie_le(jv):
            def tb(tt, acc):
                tile = d_scr[:, pl.ds(tt * _W, _W)]
                cid = jax.lax.broadcasted_iota(jnp.int32, tile.shape, 1) \
                    + tt * _W
                msk = (tile == T) & (cid <= jv)
                return acc + jnp.sum(msk.astype(jnp.int32), axis=1,
                                     keepdims=True)
            return jax.lax.fori_loop(0, NT, tb,
                                     jnp.zeros((Bq, 1), jnp.int32))

        J_all = jnp.full((Bq, 1), jnp.int32(C_pad))

        def idx_search(_):
            def b2(_, lh):
                lo, hi = lh
                mid = (lo + hi) >> 1
                p = count_tie_le(mid) >= m
                return (jnp.where(p, lo, mid + 1), jnp.where(p, mid, hi))
            nbits = max(1, (C_pad).bit_length())
            _, hi2 = jax.lax.fori_loop(
                0, nbits, b2, (jnp.zeros((Bq, 1), jnp.int32), J_all))
            return hi2

        J = jax.lax.cond(jnp.any(cnt_le > _K), idx_search,
                         lambda _: J_all, None)

        def agg(tt, acc):
            sw, swv = acc
            tile = d_scr[:, pl.ds(tt * _W, _W)]
            cid = jax.lax.broadcasted_iota(jnp.int32, tile.shape, 1) + tt * _W
            sel = (tile < T) | ((tile == T) & (cid <= J))
            w = 1.0 / (tile + jnp.float32(_DELTA))
            w = jnp.where(sel, w, 0.0)
            v = vals_ref[:, pl.ds(tt * _W, _W)]
            return (sw + jnp.sum(w, axis=1, keepdims=True),
                    swv + jnp.sum(w * v, axis=1, keepdims=True))

        z = jnp.zeros((Bq, 1), jnp.float32)
        sw, swv = jax.lax.fori_loop(0, NT, agg, (z, z))
        out_ref[...] = (swv / sw).reshape(1, 1, Bq)


def kernel(keys, dnd_keys, dnd_values):
    B, D = keys.shape
    C, _ = dnd_keys.shape
    NT = (C + _W - 1) // _W
    C_pad = NT * _W
    NB = (B + _BQ - 1) // _BQ
    B_pad = NB * _BQ

    kT = jnp.zeros((D, C_pad), jnp.float32).at[:, :C].set(dnd_keys.T)
    vals = jnp.zeros((1, C_pad), jnp.float32).at[0, :C].set(dnd_values)
    q = keys
    if B_pad != B:
        q = jnp.zeros((B_pad, D), jnp.float32).at[:B].set(keys)

    ksq = pl.pallas_call(
        functools.partial(_ksq_kernel, C=C, W=_W),
        grid=(NT,),
        in_specs=[pl.BlockSpec((D, _W), lambda j: (0, j))],
        out_specs=pl.BlockSpec((1, _W), lambda j: (0, j)),
        out_shape=jax.ShapeDtypeStruct((1, C_pad), jnp.float32),
    )(kT)

    out = pl.pallas_call(
        functools.partial(_dnd_kernel, C_pad=C_pad, NT=NT),
        grid=(NB, NT),
        in_specs=[
            pl.BlockSpec((_BQ, D), lambda i, j: (i, 0)),
            pl.BlockSpec((D, _W), lambda i, j: (0, j)),
            pl.BlockSpec((1, C_pad), lambda i, j: (0, 0)),
            pl.BlockSpec((1, C_pad), lambda i, j: (0, 0)),
        ],
        out_specs=pl.BlockSpec((1, 1, _BQ), lambda i, j: (i, 0, 0)),
        out_shape=jax.ShapeDtypeStruct((NB, 1, _BQ), jnp.float32),
        scratch_shapes=[pltpu.VMEM((_BQ, C_pad), jnp.float32),
                        pltpu.VMEM((_BQ, 1), jnp.float32)],
        compiler_params=pltpu.CompilerParams(
            dimension_semantics=("parallel", "arbitrary"),
            vmem_limit_bytes=60 * 1024 * 1024),
    )(q, kT, ksq, vals)
    return out.reshape(-1)[:B]
